# baseline (speedup 1.0000x reference)
import functools

import jax
import jax.numpy as jnp
from jax.experimental import pallas as pl
from jax.experimental.pallas import tpu as pltpu

N = 10000
D = 128
BM = 400
HALF = 200
NSTEPS = N // BM
NBUF = 3


def _copy(adj_hbm, buf, sems, step, slot, h):
    return pltpu.make_async_copy(
        adj_hbm.at[pl.ds(step * BM + h * HALF, HALF), :],
        buf.at[slot, pl.ds(h * HALF, HALF), :],
        sems.at[slot, h],
    )


def _sage_kernel(feat_blk_ref, adj_hbm, feats_ref, w1_ref, w2_ref, out_ref,
                 buf, sems):
    i = pl.program_id(0)
    slot = jax.lax.rem(i, NBUF)

    @pl.when(i == 0)
    def _():
        for s in range(NBUF - 1):
            for h in range(2):
                _copy(adj_hbm, buf, sems, s, s, h).start()

    @pl.when(i + NBUF - 1 < NSTEPS)
    def _():
        nxt = jax.lax.rem(i + NBUF - 1, NBUF)
        for h in range(2):
            _copy(adj_hbm, buf, sems, i + NBUF - 1, nxt, h).start()

    for h in range(2):
        _copy(adj_hbm, buf, sems, i, slot, h).wait()

    adj = buf[slot]
    rowsum = jnp.sum(adj, axis=1, keepdims=True)
    neigh = jnp.dot(adj, feats_ref[...], preferred_element_type=jnp.float32)
    scale = 1.0 / (rowsum + 1.0)
    out_ref[...] = (
        jnp.dot(feat_blk_ref[...], w1_ref[...], preferred_element_type=jnp.float32)
        + jnp.dot(neigh * scale, w2_ref[...], preferred_element_type=jnp.float32)
    )


@functools.partial(jax.jit, static_argnames=())
def kernel(features, adj, W):
    w1 = W[:, :D].T
    w2 = W[:, D:].T
    return pl.pallas_call(
        _sage_kernel,
        grid=(NSTEPS,),
        in_specs=[
            pl.BlockSpec((BM, D), lambda i: (i, 0)),
            pl.BlockSpec(memory_space=pltpu.MemorySpace.HBM),
            pl.BlockSpec((N, D), lambda i: (0, 0)),
            pl.BlockSpec((D, D), lambda i: (0, 0)),
            pl.BlockSpec((D, D), lambda i: (0, 0)),
        ],
        out_specs=pl.BlockSpec((BM, D), lambda i: (i, 0)),
        out_shape=jax.ShapeDtypeStruct((N, D), jnp.float32),
        scratch_shapes=[
            pltpu.VMEM((NBUF, BM, N), jnp.float32),
            pltpu.SemaphoreType.DMA((NBUF, 2)),
        ],
        compiler_params=pltpu.CompilerParams(
            dimension_semantics=("arbitrary",),
        ),
    )(features, adj, features, w1, w2)


# speedup vs baseline: 1.0237x; 1.0237x over previous
import functools

import jax
import jax.numpy as jnp
from jax.experimental import pallas as pl
from jax.experimental.pallas import tpu as pltpu

N = 10000
D = 128
BM = 400
KSPLIT = 4992
KSIZES = (KSPLIT, N - KSPLIT)
NSTEPS = N // BM
NBUF = 2


def _copy(adj_hbm, buf, sems, step, slot, h):
    k0 = 0 if h == 0 else KSPLIT
    return pltpu.make_async_copy(
        adj_hbm.at[pl.ds(step * BM, BM), pl.ds(k0, KSIZES[h])],
        buf.at[slot, :, pl.ds(k0, KSIZES[h])],
        sems.at[slot, h],
    )


def _sage_kernel(feat_blk_ref, adj_hbm, feats_ref, w1_ref, w2_ref, out_ref,
                 buf, sems):
    i = pl.program_id(0)
    slot = jax.lax.rem(i, NBUF)
    nxt = jax.lax.rem(i + 1, NBUF)

    @pl.when(i == 0)
    def _():
        for h in range(2):
            _copy(adj_hbm, buf, sems, 0, 0, h).start()

    @pl.when(i + 1 < NSTEPS)
    def _():
        for h in range(2):
            _copy(adj_hbm, buf, sems, i + 1, nxt, h).start()

    acc = jnp.zeros((BM, D), jnp.float32)
    rowsum = jnp.zeros((BM, 1), jnp.float32)
    k0 = 0
    for h in range(2):
        _copy(adj_hbm, buf, sems, i, slot, h).wait()
        adj_h = buf[slot, :, k0:k0 + KSIZES[h]]
        rowsum = rowsum + jnp.sum(adj_h, axis=1, keepdims=True)
        acc = acc + jnp.dot(adj_h, feats_ref[k0:k0 + KSIZES[h], :],
                            preferred_element_type=jnp.float32)
        k0 += KSIZES[h]

    scale = 1.0 / (rowsum + 1.0)
    out_ref[...] = (
        jnp.dot(feat_blk_ref[...], w1_ref[...], preferred_element_type=jnp.float32)
        + jnp.dot(acc * scale, w2_ref[...], preferred_element_type=jnp.float32)
    )


@functools.partial(jax.jit, static_argnames=())
def kernel(features, adj, W):
    w1 = W[:, :D].T
    w2 = W[:, D:].T
    return pl.pallas_call(
        _sage_kernel,
        grid=(NSTEPS,),
        in_specs=[
            pl.BlockSpec((BM, D), lambda i: (i, 0)),
            pl.BlockSpec(memory_space=pltpu.MemorySpace.HBM),
            pl.BlockSpec((N, D), lambda i: (0, 0)),
            pl.BlockSpec((D, D), lambda i: (0, 0)),
            pl.BlockSpec((D, D), lambda i: (0, 0)),
        ],
        out_specs=pl.BlockSpec((BM, D), lambda i: (i, 0)),
        out_shape=jax.ShapeDtypeStruct((N, D), jnp.float32),
        scratch_shapes=[
            pltpu.VMEM((NBUF, BM, N), jnp.float32),
            pltpu.SemaphoreType.DMA((NBUF, 2)),
        ],
        compiler_params=pltpu.CompilerParams(
            dimension_semantics=("arbitrary",),
        ),
    )(features, adj, features, w1, w2)
